# baseline (speedup 1.0000x reference)
import functools

import jax
import jax.numpy as jnp
from jax import lax
from jax.experimental import pallas as pl
from jax.experimental.pallas import tpu as pltpu
from jax.experimental.pallas import tpu_sc as plsc

N_NODES = 10000
N_EDGES = 160000
D_FEAT = 256
HALF = 128
N_GRAPHS = 64
N_CLASSES = 40

NP = 10240
EP = 163840
EROWS = EP // 128
ROWS_PER_TILE = EROWS // 16
CHUNK = 128
NODES_PER_TILE = NP // 16
PHASE_ROWS = ROWS_PER_TILE // 2
DUMMY = N_NODES

@functools.cache
def _sc_kernels():
    mesh = plsc.VectorSubcoreMesh(core_axis_name="c", subcore_axis_name="s")
    count_k = functools.partial(
        pl.kernel,
        out_type=jax.ShapeDtypeStruct((NP, HALF), jnp.float32),
        mesh=mesh,
        scratch_types=[
            pltpu.VMEM_SHARED((NP, HALF), jnp.float32),
            pltpu.VMEM((ROWS_PER_TILE, CHUNK), jnp.int32),
            pltpu.VMEM((CHUNK, HALF), jnp.float32),
        ],
    )(_count_body)
    agg_k = functools.partial(
        pl.kernel,
        out_type=jax.ShapeDtypeStruct((2, NP, HALF), jnp.float32),
        mesh=mesh,
        scratch_types=[
            pltpu.VMEM_SHARED((NP, HALF), jnp.float32),
            pltpu.VMEM((PHASE_ROWS, CHUNK), jnp.int32),
            pltpu.VMEM((PHASE_ROWS, CHUNK), jnp.int32),
            pltpu.VMEM((CHUNK, HALF), jnp.float32),
            pltpu.VMEM((CHUNK, HALF), jnp.float32),
            pltpu.SemaphoreType.DMA,
            pltpu.SemaphoreType.DMA,
            pltpu.SemaphoreType.DMA,
            pltpu.SemaphoreType.DMA,
        ],
    )(_agg_body)
    return count_k, agg_k


def _count_body(dst_hbm, ones_hbm, zeros_hbm, out_hbm, cnt_sh, dstb, ones_v):
    c = lax.axis_index("c")
    s = lax.axis_index("s")
    pltpu.sync_copy(ones_hbm, ones_v)
    pltpu.sync_copy(zeros_hbm, cnt_sh.at[pl.ds(s * NODES_PER_TILE, NODES_PER_TILE)])
    pltpu.sync_copy(dst_hbm.at[pl.ds(s * ROWS_PER_TILE, ROWS_PER_TILE)], dstb)
    plsc.subcore_barrier()

    def body(j, _):
        pltpu.sync_copy(ones_v, cnt_sh.at[dstb.at[j]], add=True)
        return _

    lax.fori_loop(0, ROWS_PER_TILE, body, None)
    plsc.subcore_barrier()

    @pl.when(c == 0)
    def _():
        pltpu.sync_copy(
            cnt_sh.at[pl.ds(s * NODES_PER_TILE, NODES_PER_TILE)],
            out_hbm.at[pl.ds(s * NODES_PER_TILE, NODES_PER_TILE)],
        )


def _agg_body(hl_hbm, src_hbm, dst_hbm, zeros_hbm, out_hbm,
              acc_sh, srcb, dstb, gbuf0, gbuf1, sem0, sem1, sems0, sems1):
    c = lax.axis_index("c")
    s = lax.axis_index("s")
    pltpu.sync_copy(zeros_hbm, acc_sh.at[pl.ds(s * NODES_PER_TILE, NODES_PER_TILE)])
    plsc.subcore_barrier()

    H = CHUNK // 2

    def issue(j, buf, sem):
        pltpu.async_copy(hl_hbm.at[srcb.at[j, pl.ds(0, H)]],
                         buf.at[pl.ds(0, H)], sem)
        pltpu.async_copy(hl_hbm.at[srcb.at[j, pl.ds(H, H)]],
                         buf.at[pl.ds(H, H)], sem)

    def drain(buf, sem):
        pltpu.make_async_copy(hl_hbm.at[pl.ds(0, H)], buf.at[pl.ds(0, H)], sem).wait()
        pltpu.make_async_copy(hl_hbm.at[pl.ds(0, H)], buf.at[pl.ds(H, H)], sem).wait()

    def drain_scatter(buf, sem):
        pltpu.make_async_copy(hl_hbm.at[pl.ds(0, CHUNK)], buf, sem).wait()

    def phase(base):
        pltpu.sync_copy(
            src_hbm.at[c, pl.ds(s * ROWS_PER_TILE + base, PHASE_ROWS)], srcb)
        pltpu.sync_copy(
            dst_hbm.at[pl.ds(s * ROWS_PER_TILE + base, PHASE_ROWS)], dstb)
        issue(0, gbuf0, sem0)
        issue(1, gbuf1, sem1)

        def body(i, _):
            j0 = 2 * i
            drain(gbuf0, sem0)
            pltpu.async_copy(gbuf0, acc_sh.at[dstb.at[j0]], sems0, add=True)
            drain(gbuf1, sem1)
            pltpu.async_copy(gbuf1, acc_sh.at[dstb.at[j0 + 1]], sems1, add=True)
            drain_scatter(gbuf0, sems0)

            @pl.when(i < PHASE_ROWS // 2 - 1)
            def _():
                issue(j0 + 2, gbuf0, sem0)

            drain_scatter(gbuf1, sems1)

            @pl.when(i < PHASE_ROWS // 2 - 1)
            def _():
                issue(j0 + 3, gbuf1, sem1)

            return _

        lax.fori_loop(0, PHASE_ROWS // 2, body, None)

    phase(0)
    phase(PHASE_ROWS)
    plsc.subcore_barrier()

    @pl.when(c == 0)
    def _():
        pltpu.sync_copy(
            acc_sh.at[pl.ds(s * NODES_PER_TILE, NODES_PER_TILE)],
            out_hbm.at[0, pl.ds(s * NODES_PER_TILE, NODES_PER_TILE)],
        )

    @pl.when(c == 1)
    def _():
        pltpu.sync_copy(
            acc_sh.at[pl.ds(s * NODES_PER_TILE, NODES_PER_TILE)],
            out_hbm.at[1, pl.ds(s * NODES_PER_TILE, NODES_PER_TILE)],
        )


_BLK = 128
_GRID = NP // _BLK


def _pre_body(x_ref, w_ref, hl_ref, hr_ref):
    y = jnp.dot(x_ref[...], w_ref[...], preferred_element_type=jnp.float32)
    hl_ref[0] = y[:, :HALF]
    hl_ref[1] = y[:, HALF:D_FEAT]
    hr_ref[...] = y[:, D_FEAT:]


_pre_call = pl.pallas_call(
    _pre_body,
    grid=(_GRID,),
    in_specs=[
        pl.BlockSpec((_BLK, D_FEAT), lambda i: (i, 0)),
        pl.BlockSpec((D_FEAT, 2 * D_FEAT), lambda i: (0, 0)),
    ],
    out_specs=[
        pl.BlockSpec((2, _BLK, HALF), lambda i: (0, i, 0)),
        pl.BlockSpec((_BLK, D_FEAT), lambda i: (i, 0)),
    ],
    out_shape=[
        jax.ShapeDtypeStruct((2, NP, HALF), jnp.float32),
        jax.ShapeDtypeStruct((NP, D_FEAT), jnp.float32),
    ],
)


def _mid_body(agg_ref, cnt_ref, hrp_ref, w_ref, hl_ref, hr_ref):
    inv = 1.0 / jnp.clip(cnt_ref[:, 0:1], 1.0, None)
    mean = jnp.concatenate([agg_ref[0], agg_ref[1]], axis=1) * inv
    h = jnp.maximum(mean + hrp_ref[...], 0.0)
    y = jnp.dot(h, w_ref[...], preferred_element_type=jnp.float32)
    hl_ref[0] = y[:, :HALF]
    hl_ref[1] = y[:, HALF:D_FEAT]
    hr_ref[...] = y[:, D_FEAT:]


_mid_call = pl.pallas_call(
    _mid_body,
    grid=(_GRID,),
    in_specs=[
        pl.BlockSpec((2, _BLK, HALF), lambda i: (0, i, 0)),
        pl.BlockSpec((_BLK, HALF), lambda i: (i, 0)),
        pl.BlockSpec((_BLK, D_FEAT), lambda i: (i, 0)),
        pl.BlockSpec((D_FEAT, 2 * D_FEAT), lambda i: (0, 0)),
    ],
    out_specs=[
        pl.BlockSpec((2, _BLK, HALF), lambda i: (0, i, 0)),
        pl.BlockSpec((_BLK, D_FEAT), lambda i: (i, 0)),
    ],
    out_shape=[
        jax.ShapeDtypeStruct((2, NP, HALF), jnp.float32),
        jax.ShapeDtypeStruct((NP, D_FEAT), jnp.float32),
    ],
)


def _post_body(agg_ref, cnt_ref, hrp_ref, b_ref, w1_ref, w2_ref, bias_ref,
               out_ref, psum, pcnt):
    i = pl.program_id(0)

    @pl.when(i == 0)
    def _():
        psum[...] = jnp.zeros_like(psum)
        pcnt[...] = jnp.zeros_like(pcnt)

    inv = 1.0 / jnp.clip(cnt_ref[:, 0:1], 1.0, None)
    mean = jnp.concatenate([agg_ref[0], agg_ref[1]], axis=1) * inv
    h = jnp.maximum(mean + hrp_ref[...], 0.0)
    b = b_ref[0, 0, :]
    gids = lax.broadcasted_iota(jnp.int32, (N_GRAPHS, _BLK), 0)
    onehot = (gids == b[None, :]).astype(jnp.float32)
    psum[...] += jnp.dot(onehot, h, preferred_element_type=jnp.float32)
    pcnt[...] += jnp.sum(onehot, axis=1, keepdims=True)

    @pl.when(i == _GRID - 1)
    def _():
        pooled = psum[...] / jnp.clip(pcnt[...], 1.0, None)
        z = jnp.maximum(
            jnp.dot(pooled, w1_ref[...], preferred_element_type=jnp.float32), 0.0)
        o = jnp.dot(z, w2_ref[...], preferred_element_type=jnp.float32)
        o = o + bias_ref[0, :][None, :]
        m = jnp.max(o, axis=-1, keepdims=True)
        lse = m + jnp.log(jnp.sum(jnp.exp(o - m), axis=-1, keepdims=True))
        out_ref[...] = o - lse


_post_call = pl.pallas_call(
    _post_body,
    grid=(_GRID,),
    in_specs=[
        pl.BlockSpec((2, _BLK, HALF), lambda i: (0, i, 0)),
        pl.BlockSpec((_BLK, HALF), lambda i: (i, 0)),
        pl.BlockSpec((_BLK, D_FEAT), lambda i: (i, 0)),
        pl.BlockSpec((1, 1, _BLK), lambda i: (i, 0, 0)),
        pl.BlockSpec((D_FEAT, D_FEAT), lambda i: (0, 0)),
        pl.BlockSpec((D_FEAT, N_CLASSES), lambda i: (0, 0)),
        pl.BlockSpec((1, N_CLASSES), lambda i: (0, 0)),
    ],
    out_specs=pl.BlockSpec((N_GRAPHS, N_CLASSES), lambda i: (0, 0)),
    out_shape=jax.ShapeDtypeStruct((N_GRAPHS, N_CLASSES), jnp.float32),
    scratch_shapes=[
        pltpu.VMEM((N_GRAPHS, D_FEAT), jnp.float32),
        pltpu.VMEM((N_GRAPHS, 1), jnp.float32),
    ],
)


def kernel(x, edge_index, batch, W1l, W1r, W2l, W2r, W3l, W3r,
           Wlin1, Wlin2, blin2):
    src = edge_index[0].astype(jnp.int32)
    dst = edge_index[1].astype(jnp.int32)
    pad_e = EP - N_EDGES
    src_p = jnp.concatenate([src, jnp.zeros((pad_e,), jnp.int32)])
    dst_p = jnp.concatenate([dst, jnp.full((pad_e,), DUMMY, jnp.int32)])
    src_lo = src_p.reshape(EROWS, CHUNK)
    src2 = jnp.stack([src_lo, src_lo + NP])
    dst2d = dst_p.reshape(EROWS, CHUNK)
    zeros_blk = jnp.zeros((NODES_PER_TILE, HALF), jnp.float32)
    ones_blk = jnp.ones((CHUNK, HALF), jnp.float32)

    x_p = jnp.pad(x, ((0, NP - N_NODES), (0, 0)))
    batch_p = jnp.pad(batch.astype(jnp.int32), (0, NP - N_NODES),
                      constant_values=N_GRAPHS).reshape(_GRID, 1, _BLK)

    w1 = jnp.concatenate([W1l, W1r], axis=1)
    w2 = jnp.concatenate([W2l, W2r], axis=1)
    w3 = jnp.concatenate([W3l, W3r], axis=1)
    bias2 = blin2.reshape(1, N_CLASSES)

    count_kernel, agg_kernel = _sc_kernels()
    cnt = count_kernel(dst2d, ones_blk, zeros_blk)

    hl1, hr1 = _pre_call(x_p, w1)
    agg1 = agg_kernel(hl1.reshape(2 * NP, HALF), src2, dst2d, zeros_blk)
    hl2, hr2 = _mid_call(agg1, cnt, hr1, w2)
    agg2 = agg_kernel(hl2.reshape(2 * NP, HALF), src2, dst2d, zeros_blk)
    hl3, hr3 = _mid_call(agg2, cnt, hr2, w3)
    agg3 = agg_kernel(hl3.reshape(2 * NP, HALF), src2, dst2d, zeros_blk)

    return _post_call(agg3, cnt, hr3, batch_p, Wlin1, Wlin2, bias2)


# speedup vs baseline: 1.0661x; 1.0661x over previous
import functools

import jax
import jax.numpy as jnp
from jax import lax
from jax.experimental import pallas as pl
from jax.experimental.pallas import tpu as pltpu
from jax.experimental.pallas import tpu_sc as plsc

N_NODES = 10000
N_EDGES = 160000
D_FEAT = 256
HALF = 128
N_GRAPHS = 64
N_CLASSES = 40

NP = 10240
EP = 163840
EROWS = EP // 128
ROWS_PER_TILE = EROWS // 16
CHUNK = 128
NODES_PER_TILE = NP // 16
PHASE_ROWS = ROWS_PER_TILE // 2
DUMMY = N_NODES

@functools.cache
def _sc_kernels():
    mesh = plsc.VectorSubcoreMesh(core_axis_name="c", subcore_axis_name="s")
    count_k = functools.partial(
        pl.kernel,
        out_type=jax.ShapeDtypeStruct((NP, HALF), jnp.float32),
        mesh=mesh,
        scratch_types=[
            pltpu.VMEM_SHARED((NP, HALF), jnp.float32),
            pltpu.VMEM((ROWS_PER_TILE, CHUNK), jnp.int32),
            pltpu.VMEM((CHUNK, HALF), jnp.float32),
        ],
    )(_count_body)
    agg_k = functools.partial(
        pl.kernel,
        out_type=jax.ShapeDtypeStruct((2, NP, HALF), jnp.float32),
        mesh=mesh,
        scratch_types=[
            pltpu.VMEM_SHARED((NP, HALF), jnp.float32),
            pltpu.VMEM((PHASE_ROWS, CHUNK), jnp.int32),
            pltpu.VMEM((PHASE_ROWS, CHUNK), jnp.int32),
            pltpu.VMEM((CHUNK, HALF), jnp.float32),
            pltpu.VMEM((CHUNK, HALF), jnp.float32),
            pltpu.SemaphoreType.DMA,
            pltpu.SemaphoreType.DMA,
        ],
    )(_agg_body)
    return count_k, agg_k


def _count_body(dst_hbm, ones_hbm, zeros_hbm, out_hbm, cnt_sh, dstb, ones_v):
    c = lax.axis_index("c")
    s = lax.axis_index("s")
    pltpu.sync_copy(ones_hbm, ones_v)
    pltpu.sync_copy(zeros_hbm, cnt_sh.at[pl.ds(s * NODES_PER_TILE, NODES_PER_TILE)])
    pltpu.sync_copy(dst_hbm.at[pl.ds(s * ROWS_PER_TILE, ROWS_PER_TILE)], dstb)
    plsc.subcore_barrier()

    def body(j, _):
        pltpu.sync_copy(ones_v, cnt_sh.at[dstb.at[j]], add=True)
        return _

    lax.fori_loop(0, ROWS_PER_TILE, body, None)
    plsc.subcore_barrier()

    @pl.when(c == 0)
    def _():
        pltpu.sync_copy(
            cnt_sh.at[pl.ds(s * NODES_PER_TILE, NODES_PER_TILE)],
            out_hbm.at[pl.ds(s * NODES_PER_TILE, NODES_PER_TILE)],
        )


def _agg_body(hl_hbm, src_hbm, dst_hbm, zeros_hbm, out_hbm,
              acc_sh, srcb, dstb, gbuf0, gbuf1, sem0, sem1):
    c = lax.axis_index("c")
    s = lax.axis_index("s")
    pltpu.sync_copy(zeros_hbm, acc_sh.at[pl.ds(s * NODES_PER_TILE, NODES_PER_TILE)])
    plsc.subcore_barrier()

    H = CHUNK // 2

    def issue(j, buf, sem):
        pltpu.async_copy(hl_hbm.at[srcb.at[j, pl.ds(0, H)]],
                         buf.at[pl.ds(0, H)], sem)
        pltpu.async_copy(hl_hbm.at[srcb.at[j, pl.ds(H, H)]],
                         buf.at[pl.ds(H, H)], sem)

    def drain(buf, sem):
        pltpu.make_async_copy(hl_hbm.at[pl.ds(0, H)], buf.at[pl.ds(0, H)], sem).wait()
        pltpu.make_async_copy(hl_hbm.at[pl.ds(0, H)], buf.at[pl.ds(H, H)], sem).wait()

    def phase(base):
        pltpu.sync_copy(
            src_hbm.at[c, pl.ds(s * ROWS_PER_TILE + base, PHASE_ROWS)], srcb)
        pltpu.sync_copy(
            dst_hbm.at[pl.ds(s * ROWS_PER_TILE + base, PHASE_ROWS)], dstb)
        issue(0, gbuf0, sem0)
        issue(1, gbuf1, sem1)

        def body(i, _):
            j0 = 2 * i
            drain(gbuf0, sem0)
            pltpu.sync_copy(gbuf0, acc_sh.at[dstb.at[j0]], add=True)

            @pl.when(i < PHASE_ROWS // 2 - 1)
            def _():
                issue(j0 + 2, gbuf0, sem0)

            drain(gbuf1, sem1)
            pltpu.sync_copy(gbuf1, acc_sh.at[dstb.at[j0 + 1]], add=True)

            @pl.when(i < PHASE_ROWS // 2 - 1)
            def _():
                issue(j0 + 3, gbuf1, sem1)

            return _

        lax.fori_loop(0, PHASE_ROWS // 2, body, None)

    phase(0)
    phase(PHASE_ROWS)
    plsc.subcore_barrier()

    @pl.when(c == 0)
    def _():
        pltpu.sync_copy(
            acc_sh.at[pl.ds(s * NODES_PER_TILE, NODES_PER_TILE)],
            out_hbm.at[0, pl.ds(s * NODES_PER_TILE, NODES_PER_TILE)],
        )

    @pl.when(c == 1)
    def _():
        pltpu.sync_copy(
            acc_sh.at[pl.ds(s * NODES_PER_TILE, NODES_PER_TILE)],
            out_hbm.at[1, pl.ds(s * NODES_PER_TILE, NODES_PER_TILE)],
        )


_BLK = 128
_GRID = NP // _BLK


def _pre_body(x_ref, w_ref, hl_ref, hr_ref):
    y = jnp.dot(x_ref[...], w_ref[...], preferred_element_type=jnp.float32)
    hl_ref[0] = y[:, :HALF]
    hl_ref[1] = y[:, HALF:D_FEAT]
    hr_ref[...] = y[:, D_FEAT:]


_pre_call = pl.pallas_call(
    _pre_body,
    grid=(_GRID,),
    in_specs=[
        pl.BlockSpec((_BLK, D_FEAT), lambda i: (i, 0)),
        pl.BlockSpec((D_FEAT, 2 * D_FEAT), lambda i: (0, 0)),
    ],
    out_specs=[
        pl.BlockSpec((2, _BLK, HALF), lambda i: (0, i, 0)),
        pl.BlockSpec((_BLK, D_FEAT), lambda i: (i, 0)),
    ],
    out_shape=[
        jax.ShapeDtypeStruct((2, NP, HALF), jnp.float32),
        jax.ShapeDtypeStruct((NP, D_FEAT), jnp.float32),
    ],
)


def _mid_body(agg_ref, cnt_ref, hrp_ref, w_ref, hl_ref, hr_ref):
    inv = 1.0 / jnp.clip(cnt_ref[:, 0:1], 1.0, None)
    mean = jnp.concatenate([agg_ref[0], agg_ref[1]], axis=1) * inv
    h = jnp.maximum(mean + hrp_ref[...], 0.0)
    y = jnp.dot(h, w_ref[...], preferred_element_type=jnp.float32)
    hl_ref[0] = y[:, :HALF]
    hl_ref[1] = y[:, HALF:D_FEAT]
    hr_ref[...] = y[:, D_FEAT:]


_mid_call = pl.pallas_call(
    _mid_body,
    grid=(_GRID,),
    in_specs=[
        pl.BlockSpec((2, _BLK, HALF), lambda i: (0, i, 0)),
        pl.BlockSpec((_BLK, HALF), lambda i: (i, 0)),
        pl.BlockSpec((_BLK, D_FEAT), lambda i: (i, 0)),
        pl.BlockSpec((D_FEAT, 2 * D_FEAT), lambda i: (0, 0)),
    ],
    out_specs=[
        pl.BlockSpec((2, _BLK, HALF), lambda i: (0, i, 0)),
        pl.BlockSpec((_BLK, D_FEAT), lambda i: (i, 0)),
    ],
    out_shape=[
        jax.ShapeDtypeStruct((2, NP, HALF), jnp.float32),
        jax.ShapeDtypeStruct((NP, D_FEAT), jnp.float32),
    ],
)


def _post_body(agg_ref, cnt_ref, hrp_ref, b_ref, w1_ref, w2_ref, bias_ref,
               out_ref, psum, pcnt):
    i = pl.program_id(0)

    @pl.when(i == 0)
    def _():
        psum[...] = jnp.zeros_like(psum)
        pcnt[...] = jnp.zeros_like(pcnt)

    inv = 1.0 / jnp.clip(cnt_ref[:, 0:1], 1.0, None)
    mean = jnp.concatenate([agg_ref[0], agg_ref[1]], axis=1) * inv
    h = jnp.maximum(mean + hrp_ref[...], 0.0)
    b = b_ref[0, 0, :]
    gids = lax.broadcasted_iota(jnp.int32, (N_GRAPHS, _BLK), 0)
    onehot = (gids == b[None, :]).astype(jnp.float32)
    psum[...] += jnp.dot(onehot, h, preferred_element_type=jnp.float32)
    pcnt[...] += jnp.sum(onehot, axis=1, keepdims=True)

    @pl.when(i == _GRID - 1)
    def _():
        pooled = psum[...] / jnp.clip(pcnt[...], 1.0, None)
        z = jnp.maximum(
            jnp.dot(pooled, w1_ref[...], preferred_element_type=jnp.float32), 0.0)
        o = jnp.dot(z, w2_ref[...], preferred_element_type=jnp.float32)
        o = o + bias_ref[0, :][None, :]
        m = jnp.max(o, axis=-1, keepdims=True)
        lse = m + jnp.log(jnp.sum(jnp.exp(o - m), axis=-1, keepdims=True))
        out_ref[...] = o - lse


_post_call = pl.pallas_call(
    _post_body,
    grid=(_GRID,),
    in_specs=[
        pl.BlockSpec((2, _BLK, HALF), lambda i: (0, i, 0)),
        pl.BlockSpec((_BLK, HALF), lambda i: (i, 0)),
        pl.BlockSpec((_BLK, D_FEAT), lambda i: (i, 0)),
        pl.BlockSpec((1, 1, _BLK), lambda i: (i, 0, 0)),
        pl.BlockSpec((D_FEAT, D_FEAT), lambda i: (0, 0)),
        pl.BlockSpec((D_FEAT, N_CLASSES), lambda i: (0, 0)),
        pl.BlockSpec((1, N_CLASSES), lambda i: (0, 0)),
    ],
    out_specs=pl.BlockSpec((N_GRAPHS, N_CLASSES), lambda i: (0, 0)),
    out_shape=jax.ShapeDtypeStruct((N_GRAPHS, N_CLASSES), jnp.float32),
    scratch_shapes=[
        pltpu.VMEM((N_GRAPHS, D_FEAT), jnp.float32),
        pltpu.VMEM((N_GRAPHS, 1), jnp.float32),
    ],
)


def kernel(x, edge_index, batch, W1l, W1r, W2l, W2r, W3l, W3r,
           Wlin1, Wlin2, blin2):
    src = edge_index[0].astype(jnp.int32)
    dst = edge_index[1].astype(jnp.int32)
    pad_e = EP - N_EDGES
    src_p = jnp.concatenate([src, jnp.zeros((pad_e,), jnp.int32)])
    dst_p = jnp.concatenate([dst, jnp.full((pad_e,), DUMMY, jnp.int32)])
    src_lo = src_p.reshape(EROWS, CHUNK)
    src2 = jnp.stack([src_lo, src_lo + NP])
    dst2d = dst_p.reshape(EROWS, CHUNK)
    zeros_blk = jnp.zeros((NODES_PER_TILE, HALF), jnp.float32)
    ones_blk = jnp.ones((CHUNK, HALF), jnp.float32)

    x_p = jnp.pad(x, ((0, NP - N_NODES), (0, 0)))
    batch_p = jnp.pad(batch.astype(jnp.int32), (0, NP - N_NODES),
                      constant_values=N_GRAPHS).reshape(_GRID, 1, _BLK)

    w1 = jnp.concatenate([W1l, W1r], axis=1)
    w2 = jnp.concatenate([W2l, W2r], axis=1)
    w3 = jnp.concatenate([W3l, W3r], axis=1)
    bias2 = blin2.reshape(1, N_CLASSES)

    count_kernel, agg_kernel = _sc_kernels()
    cnt = count_kernel(dst2d, ones_blk, zeros_blk)

    hl1, hr1 = _pre_call(x_p, w1)
    agg1 = agg_kernel(hl1.reshape(2 * NP, HALF), src2, dst2d, zeros_blk)
    hl2, hr2 = _mid_call(agg1, cnt, hr1, w2)
    agg2 = agg_kernel(hl2.reshape(2 * NP, HALF), src2, dst2d, zeros_blk)
    hl3, hr3 = _mid_call(agg2, cnt, hr2, w3)
    agg3 = agg_kernel(hl3.reshape(2 * NP, HALF), src2, dst2d, zeros_blk)

    return _post_call(agg3, cnt, hr3, batch_p, Wlin1, Wlin2, bias2)


# speedup vs baseline: 1.1509x; 1.0795x over previous
import functools

import jax
import jax.numpy as jnp
from jax import lax
from jax.experimental import pallas as pl
from jax.experimental.pallas import tpu as pltpu
from jax.experimental.pallas import tpu_sc as plsc

N_NODES = 10000
N_EDGES = 160000
D_FEAT = 256
HALF = 128
N_GRAPHS = 64
N_CLASSES = 40

NP = 10240
EP = 163840
EROWS = EP // 128
ROWS_PER_TILE = EROWS // 16
CHUNK = 128
NODES_PER_TILE = NP // 16
PHASE_ROWS = ROWS_PER_TILE // 2
DUMMY = N_NODES

@functools.cache
def _sc_kernels():
    mesh = plsc.VectorSubcoreMesh(core_axis_name="c", subcore_axis_name="s")
    count_k = functools.partial(
        pl.kernel,
        out_type=jax.ShapeDtypeStruct((NP, HALF), jnp.float32),
        mesh=mesh,
        scratch_types=[
            pltpu.VMEM_SHARED((NP, HALF), jnp.float32),
            pltpu.VMEM((ROWS_PER_TILE, CHUNK), jnp.int32),
            pltpu.VMEM((CHUNK, HALF), jnp.float32),
        ],
    )(_count_body)
    agg_k = functools.partial(
        pl.kernel,
        out_type=jax.ShapeDtypeStruct((2, NP, HALF), jnp.float32),
        mesh=mesh,
        scratch_types=[
            pltpu.VMEM_SHARED((NP, HALF), jnp.float32),
            pltpu.VMEM((PHASE_ROWS, CHUNK), jnp.int32),
            pltpu.VMEM((PHASE_ROWS, CHUNK), jnp.int32),
            pltpu.VMEM((CHUNK, HALF), jnp.float32),
            pltpu.VMEM((CHUNK, HALF), jnp.float32),
            pltpu.SemaphoreType.DMA,
            pltpu.SemaphoreType.DMA,
        ],
    )(_agg_body)
    return count_k, agg_k


def _count_body(dst_hbm, ones_hbm, zeros_hbm, out_hbm, cnt_sh, dstb, ones_v):
    c = lax.axis_index("c")
    s = lax.axis_index("s")
    pltpu.sync_copy(ones_hbm, ones_v)
    pltpu.sync_copy(zeros_hbm, cnt_sh.at[pl.ds(s * NODES_PER_TILE, NODES_PER_TILE)])
    pltpu.sync_copy(dst_hbm.at[pl.ds(s * ROWS_PER_TILE, ROWS_PER_TILE)], dstb)
    plsc.subcore_barrier()

    def body(j, _):
        pltpu.sync_copy(ones_v, cnt_sh.at[dstb.at[j]], add=True)
        return _

    lax.fori_loop(0, ROWS_PER_TILE, body, None)
    plsc.subcore_barrier()

    @pl.when(c == 0)
    def _():
        pltpu.sync_copy(
            cnt_sh.at[pl.ds(s * NODES_PER_TILE, NODES_PER_TILE)],
            out_hbm.at[pl.ds(s * NODES_PER_TILE, NODES_PER_TILE)],
        )


def _agg_body(hl_hbm, src_hbm, dst_hbm, zeros_hbm, out_hbm,
              acc_sh, srcb, dstb, gbuf0, gbuf1, sem0, sem1):
    c = lax.axis_index("c")
    s = lax.axis_index("s")
    pltpu.sync_copy(zeros_hbm, acc_sh.at[pl.ds(s * NODES_PER_TILE, NODES_PER_TILE)])
    plsc.subcore_barrier()

    H = CHUNK // 2

    def issue(j, buf, sem):
        pltpu.async_copy(hl_hbm.at[srcb.at[j, pl.ds(0, H)]],
                         buf.at[pl.ds(0, H)], sem)
        pltpu.async_copy(hl_hbm.at[srcb.at[j, pl.ds(H, H)]],
                         buf.at[pl.ds(H, H)], sem)

    def drain(buf, sem):
        pltpu.make_async_copy(hl_hbm.at[pl.ds(0, H)], buf.at[pl.ds(0, H)], sem).wait()
        pltpu.make_async_copy(hl_hbm.at[pl.ds(0, H)], buf.at[pl.ds(H, H)], sem).wait()

    def phase(base):
        pltpu.sync_copy(
            src_hbm.at[c, pl.ds(s * ROWS_PER_TILE + base, PHASE_ROWS)], srcb)
        pltpu.sync_copy(
            dst_hbm.at[pl.ds(s * ROWS_PER_TILE + base, PHASE_ROWS)], dstb)
        issue(0, gbuf0, sem0)
        issue(1, gbuf1, sem1)

        def body(i, _):
            j0 = 2 * i
            drain(gbuf0, sem0)
            pltpu.sync_copy(gbuf0, acc_sh.at[dstb.at[j0]], add=True)

            @pl.when(i < PHASE_ROWS // 2 - 1)
            def _():
                issue(j0 + 2, gbuf0, sem0)

            drain(gbuf1, sem1)
            pltpu.sync_copy(gbuf1, acc_sh.at[dstb.at[j0 + 1]], add=True)

            @pl.when(i < PHASE_ROWS // 2 - 1)
            def _():
                issue(j0 + 3, gbuf1, sem1)

            return _

        lax.fori_loop(0, PHASE_ROWS // 2, body, None)

    phase(0)
    phase(PHASE_ROWS)
    plsc.subcore_barrier()

    @pl.when(c == 0)
    def _():
        pltpu.sync_copy(
            acc_sh.at[pl.ds(s * NODES_PER_TILE, NODES_PER_TILE)],
            out_hbm.at[0, pl.ds(s * NODES_PER_TILE, NODES_PER_TILE)],
        )

    @pl.when(c == 1)
    def _():
        pltpu.sync_copy(
            acc_sh.at[pl.ds(s * NODES_PER_TILE, NODES_PER_TILE)],
            out_hbm.at[1, pl.ds(s * NODES_PER_TILE, NODES_PER_TILE)],
        )


_BLK = 256
_GRID = NP // _BLK


def _pre_body(x_ref, w_ref, hl_ref, hr_ref):
    y = jnp.dot(x_ref[...], w_ref[...], preferred_element_type=jnp.float32)
    hl_ref[0] = y[:, :HALF]
    hl_ref[1] = y[:, HALF:D_FEAT]
    hr_ref[...] = y[:, D_FEAT:]


_pre_call = pl.pallas_call(
    _pre_body,
    grid=(_GRID,),
    in_specs=[
        pl.BlockSpec((_BLK, D_FEAT), lambda i: (i, 0)),
        pl.BlockSpec((D_FEAT, 2 * D_FEAT), lambda i: (0, 0)),
    ],
    out_specs=[
        pl.BlockSpec((2, _BLK, HALF), lambda i: (0, i, 0)),
        pl.BlockSpec((_BLK, D_FEAT), lambda i: (i, 0)),
    ],
    out_shape=[
        jax.ShapeDtypeStruct((2, NP, HALF), jnp.float32),
        jax.ShapeDtypeStruct((NP, D_FEAT), jnp.float32),
    ],
)


def _mid_body(agg_ref, cnt_ref, hrp_ref, w_ref, hl_ref, hr_ref):
    inv = 1.0 / jnp.clip(cnt_ref[:, 0:1], 1.0, None)
    mean = jnp.concatenate([agg_ref[0], agg_ref[1]], axis=1) * inv
    h = jnp.maximum(mean + hrp_ref[...], 0.0)
    y = jnp.dot(h, w_ref[...], preferred_element_type=jnp.float32)
    hl_ref[0] = y[:, :HALF]
    hl_ref[1] = y[:, HALF:D_FEAT]
    hr_ref[...] = y[:, D_FEAT:]


_mid_call = pl.pallas_call(
    _mid_body,
    grid=(_GRID,),
    in_specs=[
        pl.BlockSpec((2, _BLK, HALF), lambda i: (0, i, 0)),
        pl.BlockSpec((_BLK, HALF), lambda i: (i, 0)),
        pl.BlockSpec((_BLK, D_FEAT), lambda i: (i, 0)),
        pl.BlockSpec((D_FEAT, 2 * D_FEAT), lambda i: (0, 0)),
    ],
    out_specs=[
        pl.BlockSpec((2, _BLK, HALF), lambda i: (0, i, 0)),
        pl.BlockSpec((_BLK, D_FEAT), lambda i: (i, 0)),
    ],
    out_shape=[
        jax.ShapeDtypeStruct((2, NP, HALF), jnp.float32),
        jax.ShapeDtypeStruct((NP, D_FEAT), jnp.float32),
    ],
)


def _post_body(agg_ref, cnt_ref, hrp_ref, b_ref, w1_ref, w2_ref, bias_ref,
               out_ref, psum, pcnt):
    i = pl.program_id(0)

    @pl.when(i == 0)
    def _():
        psum[...] = jnp.zeros_like(psum)
        pcnt[...] = jnp.zeros_like(pcnt)

    inv = 1.0 / jnp.clip(cnt_ref[:, 0:1], 1.0, None)
    mean = jnp.concatenate([agg_ref[0], agg_ref[1]], axis=1) * inv
    h = jnp.maximum(mean + hrp_ref[...], 0.0)
    b = b_ref[0, 0, :]
    gids = lax.broadcasted_iota(jnp.int32, (N_GRAPHS, _BLK), 0)
    onehot = (gids == b[None, :]).astype(jnp.float32)
    psum[...] += jnp.dot(onehot, h, preferred_element_type=jnp.float32)
    pcnt[...] += jnp.sum(onehot, axis=1, keepdims=True)

    @pl.when(i == _GRID - 1)
    def _():
        pooled = psum[...] / jnp.clip(pcnt[...], 1.0, None)
        z = jnp.maximum(
            jnp.dot(pooled, w1_ref[...], preferred_element_type=jnp.float32), 0.0)
        o = jnp.dot(z, w2_ref[...], preferred_element_type=jnp.float32)
        o = o + bias_ref[0, :][None, :]
        m = jnp.max(o, axis=-1, keepdims=True)
        lse = m + jnp.log(jnp.sum(jnp.exp(o - m), axis=-1, keepdims=True))
        out_ref[...] = o - lse


_post_call = pl.pallas_call(
    _post_body,
    grid=(_GRID,),
    in_specs=[
        pl.BlockSpec((2, _BLK, HALF), lambda i: (0, i, 0)),
        pl.BlockSpec((_BLK, HALF), lambda i: (i, 0)),
        pl.BlockSpec((_BLK, D_FEAT), lambda i: (i, 0)),
        pl.BlockSpec((1, 1, _BLK), lambda i: (i, 0, 0)),
        pl.BlockSpec((D_FEAT, D_FEAT), lambda i: (0, 0)),
        pl.BlockSpec((D_FEAT, N_CLASSES), lambda i: (0, 0)),
        pl.BlockSpec((1, N_CLASSES), lambda i: (0, 0)),
    ],
    out_specs=pl.BlockSpec((N_GRAPHS, N_CLASSES), lambda i: (0, 0)),
    out_shape=jax.ShapeDtypeStruct((N_GRAPHS, N_CLASSES), jnp.float32),
    scratch_shapes=[
        pltpu.VMEM((N_GRAPHS, D_FEAT), jnp.float32),
        pltpu.VMEM((N_GRAPHS, 1), jnp.float32),
    ],
)


def kernel(x, edge_index, batch, W1l, W1r, W2l, W2r, W3l, W3r,
           Wlin1, Wlin2, blin2):
    src = edge_index[0].astype(jnp.int32)
    dst = edge_index[1].astype(jnp.int32)
    pad_e = EP - N_EDGES
    src_p = jnp.concatenate([src, jnp.zeros((pad_e,), jnp.int32)])
    dst_p = jnp.concatenate([dst, jnp.full((pad_e,), DUMMY, jnp.int32)])
    src_lo = src_p.reshape(EROWS, CHUNK)
    src2 = jnp.stack([src_lo, src_lo + NP])
    dst2d = dst_p.reshape(EROWS, CHUNK)
    zeros_blk = jnp.zeros((NODES_PER_TILE, HALF), jnp.float32)
    ones_blk = jnp.ones((CHUNK, HALF), jnp.float32)

    x_p = jnp.pad(x, ((0, NP - N_NODES), (0, 0)))
    batch_p = jnp.pad(batch.astype(jnp.int32), (0, NP - N_NODES),
                      constant_values=N_GRAPHS).reshape(_GRID, 1, _BLK)

    w1 = jnp.concatenate([W1l, W1r], axis=1)
    w2 = jnp.concatenate([W2l, W2r], axis=1)
    w3 = jnp.concatenate([W3l, W3r], axis=1)
    bias2 = blin2.reshape(1, N_CLASSES)

    count_kernel, agg_kernel = _sc_kernels()
    cnt = count_kernel(dst2d, ones_blk, zeros_blk)

    hl1, hr1 = _pre_call(x_p, w1)
    agg1 = agg_kernel(hl1.reshape(2 * NP, HALF), src2, dst2d, zeros_blk)
    hl2, hr2 = _mid_call(agg1, cnt, hr1, w2)
    agg2 = agg_kernel(hl2.reshape(2 * NP, HALF), src2, dst2d, zeros_blk)
    hl3, hr3 = _mid_call(agg2, cnt, hr2, w3)
    agg3 = agg_kernel(hl3.reshape(2 * NP, HALF), src2, dst2d, zeros_blk)

    return _post_call(agg3, cnt, hr3, batch_p, Wlin1, Wlin2, bias2)


# speedup vs baseline: 1.1944x; 1.0378x over previous
import functools

import jax
import jax.numpy as jnp
from jax import lax
from jax.experimental import pallas as pl
from jax.experimental.pallas import tpu as pltpu
from jax.experimental.pallas import tpu_sc as plsc

N_NODES = 10000
N_EDGES = 160000
D_FEAT = 256
HALF = 128
N_GRAPHS = 64
N_CLASSES = 40

NP = 10240
EP = 163840
EROWS = EP // 128
ROWS_PER_TILE = EROWS // 16
CHUNK = 128
NODES_PER_TILE = NP // 16
PHASE_ROWS = ROWS_PER_TILE // 2
DUMMY = N_NODES

@functools.cache
def _sc_kernels():
    mesh = plsc.VectorSubcoreMesh(core_axis_name="c", subcore_axis_name="s")
    count_k = functools.partial(
        pl.kernel,
        out_type=jax.ShapeDtypeStruct((NP, HALF), jnp.float32),
        mesh=mesh,
        scratch_types=[
            pltpu.VMEM_SHARED((NP, HALF), jnp.float32),
            pltpu.VMEM((ROWS_PER_TILE, CHUNK), jnp.int32),
            pltpu.VMEM((CHUNK, HALF), jnp.float32),
        ],
    )(_count_body)
    agg_k = functools.partial(
        pl.kernel,
        out_type=jax.ShapeDtypeStruct((2, NP, HALF), jnp.float32),
        mesh=mesh,
        scratch_types=[
            pltpu.VMEM_SHARED((NP, HALF), jnp.float32),
            pltpu.VMEM((PHASE_ROWS, CHUNK), jnp.int32),
            pltpu.VMEM((PHASE_ROWS, CHUNK), jnp.int32),
            pltpu.VMEM((CHUNK, HALF), jnp.float32),
            pltpu.VMEM((CHUNK, HALF), jnp.float32),
            pltpu.SemaphoreType.DMA,
            pltpu.SemaphoreType.DMA,
        ],
    )(_agg_body)
    return count_k, agg_k


def _count_body(dst_hbm, ones_hbm, zeros_hbm, out_hbm, cnt_sh, dstb, ones_v):
    c = lax.axis_index("c")
    s = lax.axis_index("s")
    pltpu.sync_copy(ones_hbm, ones_v)
    pltpu.sync_copy(zeros_hbm, cnt_sh.at[pl.ds(s * NODES_PER_TILE, NODES_PER_TILE)])
    pltpu.sync_copy(dst_hbm.at[pl.ds(s * ROWS_PER_TILE, ROWS_PER_TILE)], dstb)
    plsc.subcore_barrier()

    def body(j, _):
        pltpu.sync_copy(ones_v, cnt_sh.at[dstb.at[j]], add=True)
        return _

    lax.fori_loop(0, ROWS_PER_TILE, body, None)
    plsc.subcore_barrier()

    @pl.when(c == 0)
    def _():
        pltpu.sync_copy(
            cnt_sh.at[pl.ds(s * NODES_PER_TILE, NODES_PER_TILE)],
            out_hbm.at[pl.ds(s * NODES_PER_TILE, NODES_PER_TILE)],
        )


def _agg_body(hl_hbm, src_hbm, dst_hbm, zeros_hbm, out_hbm,
              acc_sh, srcb, dstb, gbuf0, gbuf1, sem0, sem1):
    c = lax.axis_index("c")
    s = lax.axis_index("s")
    pltpu.sync_copy(zeros_hbm, acc_sh.at[pl.ds(s * NODES_PER_TILE, NODES_PER_TILE)])
    plsc.subcore_barrier()

    H = CHUNK // 2

    def issue(j, buf, sem):
        pltpu.async_copy(hl_hbm.at[srcb.at[j, pl.ds(0, H)]],
                         buf.at[pl.ds(0, H)], sem)
        pltpu.async_copy(hl_hbm.at[srcb.at[j, pl.ds(H, H)]],
                         buf.at[pl.ds(H, H)], sem)

    def drain(buf, sem):
        pltpu.make_async_copy(hl_hbm.at[pl.ds(0, H)], buf.at[pl.ds(0, H)], sem).wait()
        pltpu.make_async_copy(hl_hbm.at[pl.ds(0, H)], buf.at[pl.ds(H, H)], sem).wait()

    def phase(base):
        pltpu.sync_copy(
            src_hbm.at[c, pl.ds(s * ROWS_PER_TILE + base, PHASE_ROWS)], srcb)
        pltpu.sync_copy(
            dst_hbm.at[pl.ds(s * ROWS_PER_TILE + base, PHASE_ROWS)], dstb)
        issue(0, gbuf0, sem0)
        issue(1, gbuf1, sem1)

        def body(i, _):
            j0 = 2 * i
            drain(gbuf0, sem0)
            pltpu.sync_copy(gbuf0, acc_sh.at[dstb.at[j0]], add=True)

            @pl.when(i < PHASE_ROWS // 2 - 1)
            def _():
                issue(j0 + 2, gbuf0, sem0)

            drain(gbuf1, sem1)
            pltpu.sync_copy(gbuf1, acc_sh.at[dstb.at[j0 + 1]], add=True)

            @pl.when(i < PHASE_ROWS // 2 - 1)
            def _():
                issue(j0 + 3, gbuf1, sem1)

            return _

        lax.fori_loop(0, PHASE_ROWS // 2, body, None)

    phase(0)
    phase(PHASE_ROWS)
    plsc.subcore_barrier()

    @pl.when(c == 0)
    def _():
        pltpu.sync_copy(
            acc_sh.at[pl.ds(s * NODES_PER_TILE, NODES_PER_TILE)],
            out_hbm.at[0, pl.ds(s * NODES_PER_TILE, NODES_PER_TILE)],
        )

    @pl.when(c == 1)
    def _():
        pltpu.sync_copy(
            acc_sh.at[pl.ds(s * NODES_PER_TILE, NODES_PER_TILE)],
            out_hbm.at[1, pl.ds(s * NODES_PER_TILE, NODES_PER_TILE)],
        )


_BLK = 512
_GRID = NP // _BLK


def _pre_body(x_ref, w_ref, hl_ref, hr_ref):
    y = jnp.dot(x_ref[...], w_ref[...], preferred_element_type=jnp.float32)
    hl_ref[0] = y[:, :HALF]
    hl_ref[1] = y[:, HALF:D_FEAT]
    hr_ref[...] = y[:, D_FEAT:]


_pre_call = pl.pallas_call(
    _pre_body,
    grid=(_GRID,),
    in_specs=[
        pl.BlockSpec((_BLK, D_FEAT), lambda i: (i, 0)),
        pl.BlockSpec((D_FEAT, 2 * D_FEAT), lambda i: (0, 0)),
    ],
    out_specs=[
        pl.BlockSpec((2, _BLK, HALF), lambda i: (0, i, 0)),
        pl.BlockSpec((_BLK, D_FEAT), lambda i: (i, 0)),
    ],
    out_shape=[
        jax.ShapeDtypeStruct((2, NP, HALF), jnp.float32),
        jax.ShapeDtypeStruct((NP, D_FEAT), jnp.float32),
    ],
)


def _mid_body(agg_ref, cnt_ref, hrp_ref, w_ref, hl_ref, hr_ref):
    inv = 1.0 / jnp.clip(cnt_ref[:, 0:1], 1.0, None)
    mean = jnp.concatenate([agg_ref[0], agg_ref[1]], axis=1) * inv
    h = jnp.maximum(mean + hrp_ref[...], 0.0)
    y = jnp.dot(h, w_ref[...], preferred_element_type=jnp.float32)
    hl_ref[0] = y[:, :HALF]
    hl_ref[1] = y[:, HALF:D_FEAT]
    hr_ref[...] = y[:, D_FEAT:]


_mid_call = pl.pallas_call(
    _mid_body,
    grid=(_GRID,),
    in_specs=[
        pl.BlockSpec((2, _BLK, HALF), lambda i: (0, i, 0)),
        pl.BlockSpec((_BLK, HALF), lambda i: (i, 0)),
        pl.BlockSpec((_BLK, D_FEAT), lambda i: (i, 0)),
        pl.BlockSpec((D_FEAT, 2 * D_FEAT), lambda i: (0, 0)),
    ],
    out_specs=[
        pl.BlockSpec((2, _BLK, HALF), lambda i: (0, i, 0)),
        pl.BlockSpec((_BLK, D_FEAT), lambda i: (i, 0)),
    ],
    out_shape=[
        jax.ShapeDtypeStruct((2, NP, HALF), jnp.float32),
        jax.ShapeDtypeStruct((NP, D_FEAT), jnp.float32),
    ],
)


def _post_body(agg_ref, cnt_ref, hrp_ref, b_ref, w1_ref, w2_ref, bias_ref,
               out_ref, psum, pcnt):
    i = pl.program_id(0)

    @pl.when(i == 0)
    def _():
        psum[...] = jnp.zeros_like(psum)
        pcnt[...] = jnp.zeros_like(pcnt)

    inv = 1.0 / jnp.clip(cnt_ref[:, 0:1], 1.0, None)
    mean = jnp.concatenate([agg_ref[0], agg_ref[1]], axis=1) * inv
    h = jnp.maximum(mean + hrp_ref[...], 0.0)
    b = b_ref[0, 0, :]
    gids = lax.broadcasted_iota(jnp.int32, (N_GRAPHS, _BLK), 0)
    onehot = (gids == b[None, :]).astype(jnp.float32)
    psum[...] += jnp.dot(onehot, h, preferred_element_type=jnp.float32)
    pcnt[...] += jnp.sum(onehot, axis=1, keepdims=True)

    @pl.when(i == _GRID - 1)
    def _():
        pooled = psum[...] / jnp.clip(pcnt[...], 1.0, None)
        z = jnp.maximum(
            jnp.dot(pooled, w1_ref[...], preferred_element_type=jnp.float32), 0.0)
        o = jnp.dot(z, w2_ref[...], preferred_element_type=jnp.float32)
        o = o + bias_ref[0, :][None, :]
        m = jnp.max(o, axis=-1, keepdims=True)
        lse = m + jnp.log(jnp.sum(jnp.exp(o - m), axis=-1, keepdims=True))
        out_ref[...] = o - lse


_post_call = pl.pallas_call(
    _post_body,
    grid=(_GRID,),
    in_specs=[
        pl.BlockSpec((2, _BLK, HALF), lambda i: (0, i, 0)),
        pl.BlockSpec((_BLK, HALF), lambda i: (i, 0)),
        pl.BlockSpec((_BLK, D_FEAT), lambda i: (i, 0)),
        pl.BlockSpec((1, 1, _BLK), lambda i: (i, 0, 0)),
        pl.BlockSpec((D_FEAT, D_FEAT), lambda i: (0, 0)),
        pl.BlockSpec((D_FEAT, N_CLASSES), lambda i: (0, 0)),
        pl.BlockSpec((1, N_CLASSES), lambda i: (0, 0)),
    ],
    out_specs=pl.BlockSpec((N_GRAPHS, N_CLASSES), lambda i: (0, 0)),
    out_shape=jax.ShapeDtypeStruct((N_GRAPHS, N_CLASSES), jnp.float32),
    scratch_shapes=[
        pltpu.VMEM((N_GRAPHS, D_FEAT), jnp.float32),
        pltpu.VMEM((N_GRAPHS, 1), jnp.float32),
    ],
)


def kernel(x, edge_index, batch, W1l, W1r, W2l, W2r, W3l, W3r,
           Wlin1, Wlin2, blin2):
    src = edge_index[0].astype(jnp.int32)
    dst = edge_index[1].astype(jnp.int32)
    pad_e = EP - N_EDGES
    src_p = jnp.concatenate([src, jnp.zeros((pad_e,), jnp.int32)])
    dst_p = jnp.concatenate([dst, jnp.full((pad_e,), DUMMY, jnp.int32)])
    src_lo = src_p.reshape(EROWS, CHUNK)
    src2 = jnp.stack([src_lo, src_lo + NP])
    dst2d = dst_p.reshape(EROWS, CHUNK)
    zeros_blk = jnp.zeros((NODES_PER_TILE, HALF), jnp.float32)
    ones_blk = jnp.ones((CHUNK, HALF), jnp.float32)

    x_p = jnp.pad(x, ((0, NP - N_NODES), (0, 0)))
    batch_p = jnp.pad(batch.astype(jnp.int32), (0, NP - N_NODES),
                      constant_values=N_GRAPHS).reshape(_GRID, 1, _BLK)

    w1 = jnp.concatenate([W1l, W1r], axis=1)
    w2 = jnp.concatenate([W2l, W2r], axis=1)
    w3 = jnp.concatenate([W3l, W3r], axis=1)
    bias2 = blin2.reshape(1, N_CLASSES)

    count_kernel, agg_kernel = _sc_kernels()
    cnt = count_kernel(dst2d, ones_blk, zeros_blk)

    hl1, hr1 = _pre_call(x_p, w1)
    agg1 = agg_kernel(hl1.reshape(2 * NP, HALF), src2, dst2d, zeros_blk)
    hl2, hr2 = _mid_call(agg1, cnt, hr1, w2)
    agg2 = agg_kernel(hl2.reshape(2 * NP, HALF), src2, dst2d, zeros_blk)
    hl3, hr3 = _mid_call(agg2, cnt, hr2, w3)
    agg3 = agg_kernel(hl3.reshape(2 * NP, HALF), src2, dst2d, zeros_blk)

    return _post_call(agg3, cnt, hr3, batch_p, Wlin1, Wlin2, bias2)


# speedup vs baseline: 1.2226x; 1.0236x over previous
import functools

import jax
import jax.numpy as jnp
from jax import lax
from jax.experimental import pallas as pl
from jax.experimental.pallas import tpu as pltpu
from jax.experimental.pallas import tpu_sc as plsc

N_NODES = 10000
N_EDGES = 160000
D_FEAT = 256
HALF = 128
N_GRAPHS = 64
N_CLASSES = 40

NP = 10240
EP = 163840
EROWS = EP // 128
ROWS_PER_TILE = EROWS // 16
CHUNK = 128
NODES_PER_TILE = NP // 16
PHASE_ROWS = ROWS_PER_TILE // 2
DUMMY = N_NODES

@functools.cache
def _sc_kernels():
    mesh = plsc.VectorSubcoreMesh(core_axis_name="c", subcore_axis_name="s")
    count_k = functools.partial(
        pl.kernel,
        out_type=jax.ShapeDtypeStruct((NP, HALF), jnp.float32),
        mesh=mesh,
        scratch_types=[
            pltpu.VMEM_SHARED((NP, HALF), jnp.float32),
            pltpu.VMEM((ROWS_PER_TILE, CHUNK), jnp.int32),
            pltpu.VMEM((CHUNK, HALF), jnp.float32),
        ],
    )(_count_body)
    agg_k = functools.partial(
        pl.kernel,
        out_type=jax.ShapeDtypeStruct((2, NP, HALF), jnp.float32),
        mesh=mesh,
        scratch_types=[
            pltpu.VMEM_SHARED((NP, HALF), jnp.float32),
            pltpu.VMEM((PHASE_ROWS, CHUNK), jnp.int32),
            pltpu.VMEM((PHASE_ROWS, CHUNK), jnp.int32),
            pltpu.VMEM((CHUNK, HALF), jnp.float32),
            pltpu.VMEM((CHUNK, HALF), jnp.float32),
            pltpu.SemaphoreType.DMA,
            pltpu.SemaphoreType.DMA,
        ],
    )(_agg_body)
    return count_k, agg_k


def _count_body(dst_hbm, ones_hbm, zeros_hbm, out_hbm, cnt_sh, dstb, ones_v):
    c = lax.axis_index("c")
    s = lax.axis_index("s")
    pltpu.sync_copy(ones_hbm, ones_v)
    pltpu.sync_copy(zeros_hbm, cnt_sh.at[pl.ds(s * NODES_PER_TILE, NODES_PER_TILE)])
    pltpu.sync_copy(dst_hbm.at[pl.ds(s * ROWS_PER_TILE, ROWS_PER_TILE)], dstb)
    plsc.subcore_barrier()

    def body(j, _):
        pltpu.sync_copy(ones_v, cnt_sh.at[dstb.at[j]], add=True)
        return _

    lax.fori_loop(0, ROWS_PER_TILE, body, None)
    plsc.subcore_barrier()

    @pl.when(c == 0)
    def _():
        pltpu.sync_copy(
            cnt_sh.at[pl.ds(s * NODES_PER_TILE, NODES_PER_TILE)],
            out_hbm.at[pl.ds(s * NODES_PER_TILE, NODES_PER_TILE)],
        )


def _agg_body(hl_hbm, src_hbm, dst_hbm, zeros_hbm, out_hbm,
              acc_sh, srcb, dstb, gbuf0, gbuf1, sem0, sem1):
    c = lax.axis_index("c")
    s = lax.axis_index("s")
    pltpu.sync_copy(zeros_hbm, acc_sh.at[pl.ds(s * NODES_PER_TILE, NODES_PER_TILE)])
    plsc.subcore_barrier()

    H = CHUNK // 2

    def issue(j, buf, sem):
        pltpu.async_copy(hl_hbm.at[srcb.at[j, pl.ds(0, H)]],
                         buf.at[pl.ds(0, H)], sem)
        pltpu.async_copy(hl_hbm.at[srcb.at[j, pl.ds(H, H)]],
                         buf.at[pl.ds(H, H)], sem)

    def drain(buf, sem):
        pltpu.make_async_copy(hl_hbm.at[pl.ds(0, H)], buf.at[pl.ds(0, H)], sem).wait()
        pltpu.make_async_copy(hl_hbm.at[pl.ds(0, H)], buf.at[pl.ds(H, H)], sem).wait()

    def phase(base):
        pltpu.sync_copy(
            src_hbm.at[c, pl.ds(s * ROWS_PER_TILE + base, PHASE_ROWS)], srcb)
        pltpu.sync_copy(
            dst_hbm.at[pl.ds(s * ROWS_PER_TILE + base, PHASE_ROWS)], dstb)
        issue(0, gbuf0, sem0)
        issue(1, gbuf1, sem1)

        def body(i, _):
            j0 = 2 * i
            drain(gbuf0, sem0)
            pltpu.sync_copy(gbuf0, acc_sh.at[dstb.at[j0]], add=True)

            @pl.when(i < PHASE_ROWS // 2 - 1)
            def _():
                issue(j0 + 2, gbuf0, sem0)

            drain(gbuf1, sem1)
            pltpu.sync_copy(gbuf1, acc_sh.at[dstb.at[j0 + 1]], add=True)

            @pl.when(i < PHASE_ROWS // 2 - 1)
            def _():
                issue(j0 + 3, gbuf1, sem1)

            return _

        lax.fori_loop(0, PHASE_ROWS // 2, body, None)

    phase(0)
    phase(PHASE_ROWS)
    plsc.subcore_barrier()

    @pl.when(c == 0)
    def _():
        pltpu.sync_copy(
            acc_sh.at[pl.ds(s * NODES_PER_TILE, NODES_PER_TILE)],
            out_hbm.at[0, pl.ds(s * NODES_PER_TILE, NODES_PER_TILE)],
        )

    @pl.when(c == 1)
    def _():
        pltpu.sync_copy(
            acc_sh.at[pl.ds(s * NODES_PER_TILE, NODES_PER_TILE)],
            out_hbm.at[1, pl.ds(s * NODES_PER_TILE, NODES_PER_TILE)],
        )


_BLK = 1024
_GRID = NP // _BLK


def _pre_body(x_ref, w_ref, hl_ref, hr_ref):
    y = jnp.dot(x_ref[...], w_ref[...], preferred_element_type=jnp.float32)
    hl_ref[0] = y[:, :HALF]
    hl_ref[1] = y[:, HALF:D_FEAT]
    hr_ref[...] = y[:, D_FEAT:]


_pre_call = pl.pallas_call(
    _pre_body,
    grid=(_GRID,),
    in_specs=[
        pl.BlockSpec((_BLK, D_FEAT), lambda i: (i, 0)),
        pl.BlockSpec((D_FEAT, 2 * D_FEAT), lambda i: (0, 0)),
    ],
    out_specs=[
        pl.BlockSpec((2, _BLK, HALF), lambda i: (0, i, 0)),
        pl.BlockSpec((_BLK, D_FEAT), lambda i: (i, 0)),
    ],
    out_shape=[
        jax.ShapeDtypeStruct((2, NP, HALF), jnp.float32),
        jax.ShapeDtypeStruct((NP, D_FEAT), jnp.float32),
    ],
)


def _mid_body(agg_ref, cnt_ref, hrp_ref, w_ref, hl_ref, hr_ref):
    inv = 1.0 / jnp.clip(cnt_ref[:, 0:1], 1.0, None)
    mean = jnp.concatenate([agg_ref[0], agg_ref[1]], axis=1) * inv
    h = jnp.maximum(mean + hrp_ref[...], 0.0)
    y = jnp.dot(h, w_ref[...], preferred_element_type=jnp.float32)
    hl_ref[0] = y[:, :HALF]
    hl_ref[1] = y[:, HALF:D_FEAT]
    hr_ref[...] = y[:, D_FEAT:]


_mid_call = pl.pallas_call(
    _mid_body,
    grid=(_GRID,),
    in_specs=[
        pl.BlockSpec((2, _BLK, HALF), lambda i: (0, i, 0)),
        pl.BlockSpec((_BLK, HALF), lambda i: (i, 0)),
        pl.BlockSpec((_BLK, D_FEAT), lambda i: (i, 0)),
        pl.BlockSpec((D_FEAT, 2 * D_FEAT), lambda i: (0, 0)),
    ],
    out_specs=[
        pl.BlockSpec((2, _BLK, HALF), lambda i: (0, i, 0)),
        pl.BlockSpec((_BLK, D_FEAT), lambda i: (i, 0)),
    ],
    out_shape=[
        jax.ShapeDtypeStruct((2, NP, HALF), jnp.float32),
        jax.ShapeDtypeStruct((NP, D_FEAT), jnp.float32),
    ],
)


def _post_body(agg_ref, cnt_ref, hrp_ref, b_ref, w1_ref, w2_ref, bias_ref,
               out_ref, psum, pcnt):
    i = pl.program_id(0)

    @pl.when(i == 0)
    def _():
        psum[...] = jnp.zeros_like(psum)
        pcnt[...] = jnp.zeros_like(pcnt)

    inv = 1.0 / jnp.clip(cnt_ref[:, 0:1], 1.0, None)
    mean = jnp.concatenate([agg_ref[0], agg_ref[1]], axis=1) * inv
    h = jnp.maximum(mean + hrp_ref[...], 0.0)
    b = b_ref[0, 0, :]
    gids = lax.broadcasted_iota(jnp.int32, (N_GRAPHS, _BLK), 0)
    onehot = (gids == b[None, :]).astype(jnp.float32)
    psum[...] += jnp.dot(onehot, h, preferred_element_type=jnp.float32)
    pcnt[...] += jnp.sum(onehot, axis=1, keepdims=True)

    @pl.when(i == _GRID - 1)
    def _():
        pooled = psum[...] / jnp.clip(pcnt[...], 1.0, None)
        z = jnp.maximum(
            jnp.dot(pooled, w1_ref[...], preferred_element_type=jnp.float32), 0.0)
        o = jnp.dot(z, w2_ref[...], preferred_element_type=jnp.float32)
        o = o + bias_ref[0, :][None, :]
        m = jnp.max(o, axis=-1, keepdims=True)
        lse = m + jnp.log(jnp.sum(jnp.exp(o - m), axis=-1, keepdims=True))
        out_ref[...] = o - lse


_post_call = pl.pallas_call(
    _post_body,
    grid=(_GRID,),
    in_specs=[
        pl.BlockSpec((2, _BLK, HALF), lambda i: (0, i, 0)),
        pl.BlockSpec((_BLK, HALF), lambda i: (i, 0)),
        pl.BlockSpec((_BLK, D_FEAT), lambda i: (i, 0)),
        pl.BlockSpec((1, 1, _BLK), lambda i: (i, 0, 0)),
        pl.BlockSpec((D_FEAT, D_FEAT), lambda i: (0, 0)),
        pl.BlockSpec((D_FEAT, N_CLASSES), lambda i: (0, 0)),
        pl.BlockSpec((1, N_CLASSES), lambda i: (0, 0)),
    ],
    out_specs=pl.BlockSpec((N_GRAPHS, N_CLASSES), lambda i: (0, 0)),
    out_shape=jax.ShapeDtypeStruct((N_GRAPHS, N_CLASSES), jnp.float32),
    scratch_shapes=[
        pltpu.VMEM((N_GRAPHS, D_FEAT), jnp.float32),
        pltpu.VMEM((N_GRAPHS, 1), jnp.float32),
    ],
)


def kernel(x, edge_index, batch, W1l, W1r, W2l, W2r, W3l, W3r,
           Wlin1, Wlin2, blin2):
    src = edge_index[0].astype(jnp.int32)
    dst = edge_index[1].astype(jnp.int32)
    pad_e = EP - N_EDGES
    src_p = jnp.concatenate([src, jnp.zeros((pad_e,), jnp.int32)])
    dst_p = jnp.concatenate([dst, jnp.full((pad_e,), DUMMY, jnp.int32)])
    src_lo = src_p.reshape(EROWS, CHUNK)
    src2 = jnp.stack([src_lo, src_lo + NP])
    dst2d = dst_p.reshape(EROWS, CHUNK)
    zeros_blk = jnp.zeros((NODES_PER_TILE, HALF), jnp.float32)
    ones_blk = jnp.ones((CHUNK, HALF), jnp.float32)

    x_p = jnp.pad(x, ((0, NP - N_NODES), (0, 0)))
    batch_p = jnp.pad(batch.astype(jnp.int32), (0, NP - N_NODES),
                      constant_values=N_GRAPHS).reshape(_GRID, 1, _BLK)

    w1 = jnp.concatenate([W1l, W1r], axis=1)
    w2 = jnp.concatenate([W2l, W2r], axis=1)
    w3 = jnp.concatenate([W3l, W3r], axis=1)
    bias2 = blin2.reshape(1, N_CLASSES)

    count_kernel, agg_kernel = _sc_kernels()
    cnt = count_kernel(dst2d, ones_blk, zeros_blk)

    hl1, hr1 = _pre_call(x_p, w1)
    agg1 = agg_kernel(hl1.reshape(2 * NP, HALF), src2, dst2d, zeros_blk)
    hl2, hr2 = _mid_call(agg1, cnt, hr1, w2)
    agg2 = agg_kernel(hl2.reshape(2 * NP, HALF), src2, dst2d, zeros_blk)
    hl3, hr3 = _mid_call(agg2, cnt, hr2, w3)
    agg3 = agg_kernel(hl3.reshape(2 * NP, HALF), src2, dst2d, zeros_blk)

    return _post_call(agg3, cnt, hr3, batch_p, Wlin1, Wlin2, bias2)


# speedup vs baseline: 1.2329x; 1.0084x over previous
import functools

import jax
import jax.numpy as jnp
from jax import lax
from jax.experimental import pallas as pl
from jax.experimental.pallas import tpu as pltpu
from jax.experimental.pallas import tpu_sc as plsc

N_NODES = 10000
N_EDGES = 160000
D_FEAT = 256
HALF = 128
N_GRAPHS = 64
N_CLASSES = 40

NP = 10240
EP = 163840
EROWS = EP // 128
ROWS_PER_TILE = EROWS // 16
CHUNK = 128
NODES_PER_TILE = NP // 16
PHASE_ROWS = ROWS_PER_TILE // 2
DUMMY = N_NODES

@functools.cache
def _sc_kernels():
    mesh = plsc.VectorSubcoreMesh(core_axis_name="c", subcore_axis_name="s")
    count_k = functools.partial(
        pl.kernel,
        out_type=jax.ShapeDtypeStruct((NP, HALF), jnp.float32),
        mesh=mesh,
        scratch_types=[
            pltpu.VMEM_SHARED((NP, HALF), jnp.float32),
            pltpu.VMEM((ROWS_PER_TILE, CHUNK), jnp.int32),
            pltpu.VMEM((CHUNK, HALF), jnp.float32),
        ],
    )(_count_body)
    agg_k = functools.partial(
        pl.kernel,
        out_type=jax.ShapeDtypeStruct((2, NP, HALF), jnp.float32),
        mesh=mesh,
        scratch_types=[
            pltpu.VMEM_SHARED((NP, HALF), jnp.float32),
            pltpu.VMEM((PHASE_ROWS, CHUNK), jnp.int32),
            pltpu.VMEM((PHASE_ROWS, CHUNK), jnp.int32),
            pltpu.VMEM((CHUNK, HALF), jnp.float32),
            pltpu.VMEM((CHUNK, HALF), jnp.float32),
            pltpu.SemaphoreType.DMA,
            pltpu.SemaphoreType.DMA,
        ],
    )(_agg_body)
    return count_k, agg_k


def _count_body(dst_hbm, ones_hbm, zeros_hbm, out_hbm, cnt_sh, dstb, ones_v):
    c = lax.axis_index("c")
    s = lax.axis_index("s")
    pltpu.sync_copy(ones_hbm, ones_v)
    pltpu.sync_copy(zeros_hbm, cnt_sh.at[pl.ds(s * NODES_PER_TILE, NODES_PER_TILE)])
    pltpu.sync_copy(dst_hbm.at[pl.ds(s * ROWS_PER_TILE, ROWS_PER_TILE)], dstb)
    plsc.subcore_barrier()

    def body(j, _):
        pltpu.sync_copy(ones_v, cnt_sh.at[dstb.at[j]], add=True)
        return _

    lax.fori_loop(0, ROWS_PER_TILE, body, None)
    plsc.subcore_barrier()

    @pl.when(c == 0)
    def _():
        pltpu.sync_copy(
            cnt_sh.at[pl.ds(s * NODES_PER_TILE, NODES_PER_TILE)],
            out_hbm.at[pl.ds(s * NODES_PER_TILE, NODES_PER_TILE)],
        )


def _agg_body(hl_hbm, src_hbm, dst_hbm, zeros_hbm, out_hbm,
              acc_sh, srcb, dstb, gbuf0, gbuf1, sem0, sem1):
    c = lax.axis_index("c")
    s = lax.axis_index("s")
    pltpu.sync_copy(zeros_hbm, acc_sh.at[pl.ds(s * NODES_PER_TILE, NODES_PER_TILE)])
    plsc.subcore_barrier()

    H = CHUNK // 2

    def issue(j, buf, sem):
        pltpu.async_copy(hl_hbm.at[srcb.at[j, pl.ds(0, H)]],
                         buf.at[pl.ds(0, H)], sem)
        pltpu.async_copy(hl_hbm.at[srcb.at[j, pl.ds(H, H)]],
                         buf.at[pl.ds(H, H)], sem)

    def drain(buf, sem):
        pltpu.make_async_copy(hl_hbm.at[pl.ds(0, H)], buf.at[pl.ds(0, H)], sem).wait()
        pltpu.make_async_copy(hl_hbm.at[pl.ds(0, H)], buf.at[pl.ds(H, H)], sem).wait()

    def phase(base):
        pltpu.sync_copy(
            src_hbm.at[c, pl.ds(s * ROWS_PER_TILE + base, PHASE_ROWS)], srcb)
        pltpu.sync_copy(
            dst_hbm.at[pl.ds(s * ROWS_PER_TILE + base, PHASE_ROWS)], dstb)
        issue(0, gbuf0, sem0)
        issue(1, gbuf1, sem1)

        def body(i, _):
            j0 = 2 * i
            drain(gbuf0, sem0)
            pltpu.sync_copy(gbuf0, acc_sh.at[dstb.at[j0]], add=True)

            @pl.when(i < PHASE_ROWS // 2 - 1)
            def _():
                issue(j0 + 2, gbuf0, sem0)

            drain(gbuf1, sem1)
            pltpu.sync_copy(gbuf1, acc_sh.at[dstb.at[j0 + 1]], add=True)

            @pl.when(i < PHASE_ROWS // 2 - 1)
            def _():
                issue(j0 + 3, gbuf1, sem1)

            return _

        lax.fori_loop(0, PHASE_ROWS // 2, body, None)

    phase(0)
    phase(PHASE_ROWS)
    plsc.subcore_barrier()

    @pl.when(c == 0)
    def _():
        pltpu.sync_copy(
            acc_sh.at[pl.ds(s * NODES_PER_TILE, NODES_PER_TILE)],
            out_hbm.at[0, pl.ds(s * NODES_PER_TILE, NODES_PER_TILE)],
        )

    @pl.when(c == 1)
    def _():
        pltpu.sync_copy(
            acc_sh.at[pl.ds(s * NODES_PER_TILE, NODES_PER_TILE)],
            out_hbm.at[1, pl.ds(s * NODES_PER_TILE, NODES_PER_TILE)],
        )


_BLK = 2048
_GRID = NP // _BLK


def _pre_body(x_ref, w_ref, hl_ref, hr_ref):
    y = jnp.dot(x_ref[...], w_ref[...], preferred_element_type=jnp.float32)
    hl_ref[0] = y[:, :HALF]
    hl_ref[1] = y[:, HALF:D_FEAT]
    hr_ref[...] = y[:, D_FEAT:]


_pre_call = pl.pallas_call(
    _pre_body,
    grid=(_GRID,),
    in_specs=[
        pl.BlockSpec((_BLK, D_FEAT), lambda i: (i, 0)),
        pl.BlockSpec((D_FEAT, 2 * D_FEAT), lambda i: (0, 0)),
    ],
    out_specs=[
        pl.BlockSpec((2, _BLK, HALF), lambda i: (0, i, 0)),
        pl.BlockSpec((_BLK, D_FEAT), lambda i: (i, 0)),
    ],
    out_shape=[
        jax.ShapeDtypeStruct((2, NP, HALF), jnp.float32),
        jax.ShapeDtypeStruct((NP, D_FEAT), jnp.float32),
    ],
)


def _mid_body(agg_ref, cnt_ref, hrp_ref, w_ref, hl_ref, hr_ref):
    inv = 1.0 / jnp.clip(cnt_ref[:, 0:1], 1.0, None)
    mean = jnp.concatenate([agg_ref[0], agg_ref[1]], axis=1) * inv
    h = jnp.maximum(mean + hrp_ref[...], 0.0)
    y = jnp.dot(h, w_ref[...], preferred_element_type=jnp.float32)
    hl_ref[0] = y[:, :HALF]
    hl_ref[1] = y[:, HALF:D_FEAT]
    hr_ref[...] = y[:, D_FEAT:]


_mid_call = pl.pallas_call(
    _mid_body,
    grid=(_GRID,),
    in_specs=[
        pl.BlockSpec((2, _BLK, HALF), lambda i: (0, i, 0)),
        pl.BlockSpec((_BLK, HALF), lambda i: (i, 0)),
        pl.BlockSpec((_BLK, D_FEAT), lambda i: (i, 0)),
        pl.BlockSpec((D_FEAT, 2 * D_FEAT), lambda i: (0, 0)),
    ],
    out_specs=[
        pl.BlockSpec((2, _BLK, HALF), lambda i: (0, i, 0)),
        pl.BlockSpec((_BLK, D_FEAT), lambda i: (i, 0)),
    ],
    out_shape=[
        jax.ShapeDtypeStruct((2, NP, HALF), jnp.float32),
        jax.ShapeDtypeStruct((NP, D_FEAT), jnp.float32),
    ],
)


def _post_body(agg_ref, cnt_ref, hrp_ref, b_ref, w1_ref, w2_ref, bias_ref,
               out_ref, psum, pcnt):
    i = pl.program_id(0)

    @pl.when(i == 0)
    def _():
        psum[...] = jnp.zeros_like(psum)
        pcnt[...] = jnp.zeros_like(pcnt)

    inv = 1.0 / jnp.clip(cnt_ref[:, 0:1], 1.0, None)
    mean = jnp.concatenate([agg_ref[0], agg_ref[1]], axis=1) * inv
    h = jnp.maximum(mean + hrp_ref[...], 0.0)
    b = b_ref[0, 0, :]
    gids = lax.broadcasted_iota(jnp.int32, (N_GRAPHS, _BLK), 0)
    onehot = (gids == b[None, :]).astype(jnp.float32)
    psum[...] += jnp.dot(onehot, h, preferred_element_type=jnp.float32)
    pcnt[...] += jnp.sum(onehot, axis=1, keepdims=True)

    @pl.when(i == _GRID - 1)
    def _():
        pooled = psum[...] / jnp.clip(pcnt[...], 1.0, None)
        z = jnp.maximum(
            jnp.dot(pooled, w1_ref[...], preferred_element_type=jnp.float32), 0.0)
        o = jnp.dot(z, w2_ref[...], preferred_element_type=jnp.float32)
        o = o + bias_ref[0, :][None, :]
        m = jnp.max(o, axis=-1, keepdims=True)
        lse = m + jnp.log(jnp.sum(jnp.exp(o - m), axis=-1, keepdims=True))
        out_ref[...] = o - lse


_post_call = pl.pallas_call(
    _post_body,
    grid=(_GRID,),
    in_specs=[
        pl.BlockSpec((2, _BLK, HALF), lambda i: (0, i, 0)),
        pl.BlockSpec((_BLK, HALF), lambda i: (i, 0)),
        pl.BlockSpec((_BLK, D_FEAT), lambda i: (i, 0)),
        pl.BlockSpec((1, 1, _BLK), lambda i: (i, 0, 0)),
        pl.BlockSpec((D_FEAT, D_FEAT), lambda i: (0, 0)),
        pl.BlockSpec((D_FEAT, N_CLASSES), lambda i: (0, 0)),
        pl.BlockSpec((1, N_CLASSES), lambda i: (0, 0)),
    ],
    out_specs=pl.BlockSpec((N_GRAPHS, N_CLASSES), lambda i: (0, 0)),
    out_shape=jax.ShapeDtypeStruct((N_GRAPHS, N_CLASSES), jnp.float32),
    scratch_shapes=[
        pltpu.VMEM((N_GRAPHS, D_FEAT), jnp.float32),
        pltpu.VMEM((N_GRAPHS, 1), jnp.float32),
    ],
)


def kernel(x, edge_index, batch, W1l, W1r, W2l, W2r, W3l, W3r,
           Wlin1, Wlin2, blin2):
    src = edge_index[0].astype(jnp.int32)
    dst = edge_index[1].astype(jnp.int32)
    pad_e = EP - N_EDGES
    src_p = jnp.concatenate([src, jnp.zeros((pad_e,), jnp.int32)])
    dst_p = jnp.concatenate([dst, jnp.full((pad_e,), DUMMY, jnp.int32)])
    src_lo = src_p.reshape(EROWS, CHUNK)
    src2 = jnp.stack([src_lo, src_lo + NP])
    dst2d = dst_p.reshape(EROWS, CHUNK)
    zeros_blk = jnp.zeros((NODES_PER_TILE, HALF), jnp.float32)
    ones_blk = jnp.ones((CHUNK, HALF), jnp.float32)

    x_p = jnp.pad(x, ((0, NP - N_NODES), (0, 0)))
    batch_p = jnp.pad(batch.astype(jnp.int32), (0, NP - N_NODES),
                      constant_values=N_GRAPHS).reshape(_GRID, 1, _BLK)

    w1 = jnp.concatenate([W1l, W1r], axis=1)
    w2 = jnp.concatenate([W2l, W2r], axis=1)
    w3 = jnp.concatenate([W3l, W3r], axis=1)
    bias2 = blin2.reshape(1, N_CLASSES)

    count_kernel, agg_kernel = _sc_kernels()
    cnt = count_kernel(dst2d, ones_blk, zeros_blk)

    hl1, hr1 = _pre_call(x_p, w1)
    agg1 = agg_kernel(hl1.reshape(2 * NP, HALF), src2, dst2d, zeros_blk)
    hl2, hr2 = _mid_call(agg1, cnt, hr1, w2)
    agg2 = agg_kernel(hl2.reshape(2 * NP, HALF), src2, dst2d, zeros_blk)
    hl3, hr3 = _mid_call(agg2, cnt, hr2, w3)
    agg3 = agg_kernel(hl3.reshape(2 * NP, HALF), src2, dst2d, zeros_blk)

    return _post_call(agg3, cnt, hr3, batch_p, Wlin1, Wlin2, bias2)


# speedup vs baseline: 1.2372x; 1.0036x over previous
import functools

import jax
import jax.numpy as jnp
from jax import lax
from jax.experimental import pallas as pl
from jax.experimental.pallas import tpu as pltpu
from jax.experimental.pallas import tpu_sc as plsc

N_NODES = 10000
N_EDGES = 160000
D_FEAT = 256
HALF = 128
N_GRAPHS = 64
N_CLASSES = 40

NP = 10240
EP = 163840
EROWS = EP // 128
ROWS_PER_TILE = EROWS // 16
CHUNK = 128
NODES_PER_TILE = NP // 16
PHASE_ROWS = ROWS_PER_TILE // 2
DUMMY = N_NODES

@functools.cache
def _sc_kernels():
    mesh = plsc.VectorSubcoreMesh(core_axis_name="c", subcore_axis_name="s")
    count_k = functools.partial(
        pl.kernel,
        out_type=jax.ShapeDtypeStruct((NP, HALF), jnp.float32),
        mesh=mesh,
        scratch_types=[
            pltpu.VMEM_SHARED((NP, HALF), jnp.float32),
            pltpu.VMEM((ROWS_PER_TILE, CHUNK), jnp.int32),
            pltpu.VMEM((CHUNK, HALF), jnp.float32),
        ],
    )(_count_body)
    agg_k = functools.partial(
        pl.kernel,
        out_type=jax.ShapeDtypeStruct((2, NP, HALF), jnp.float32),
        mesh=mesh,
        scratch_types=[
            pltpu.VMEM_SHARED((NP, HALF), jnp.float32),
            pltpu.VMEM((PHASE_ROWS, CHUNK), jnp.int32),
            pltpu.VMEM((PHASE_ROWS, CHUNK), jnp.int32),
            pltpu.VMEM((CHUNK, HALF), jnp.float32),
            pltpu.VMEM((CHUNK, HALF), jnp.float32),
            pltpu.SemaphoreType.DMA,
            pltpu.SemaphoreType.DMA,
        ],
    )(_agg_body)
    return count_k, agg_k


def _count_body(dst_hbm, ones_hbm, zeros_hbm, out_hbm, cnt_sh, dstb, ones_v):
    c = lax.axis_index("c")
    s = lax.axis_index("s")
    pltpu.sync_copy(ones_hbm, ones_v)
    pltpu.sync_copy(zeros_hbm, cnt_sh.at[pl.ds(s * NODES_PER_TILE, NODES_PER_TILE)])
    pltpu.sync_copy(dst_hbm.at[pl.ds(s * ROWS_PER_TILE, ROWS_PER_TILE)], dstb)
    plsc.subcore_barrier()

    def body(j, _):
        pltpu.sync_copy(ones_v, cnt_sh.at[dstb.at[j]], add=True)
        return _

    lax.fori_loop(0, ROWS_PER_TILE, body, None)
    plsc.subcore_barrier()

    @pl.when(c == 0)
    def _():
        pltpu.sync_copy(
            cnt_sh.at[pl.ds(s * NODES_PER_TILE, NODES_PER_TILE)],
            out_hbm.at[pl.ds(s * NODES_PER_TILE, NODES_PER_TILE)],
        )


def _agg_body(hl_hbm, src_hbm, dst_hbm, zeros_hbm, out_hbm,
              acc_sh, srcb, dstb, gbuf0, gbuf1, sem0, sem1):
    c = lax.axis_index("c")
    s = lax.axis_index("s")
    pltpu.sync_copy(zeros_hbm, acc_sh.at[pl.ds(s * NODES_PER_TILE, NODES_PER_TILE)])
    plsc.subcore_barrier()

    H = CHUNK // 2

    def issue(j, buf, sem):
        pltpu.async_copy(hl_hbm.at[srcb.at[j, pl.ds(0, H)]],
                         buf.at[pl.ds(0, H)], sem)
        pltpu.async_copy(hl_hbm.at[srcb.at[j, pl.ds(H, H)]],
                         buf.at[pl.ds(H, H)], sem)

    def drain(buf, sem):
        pltpu.make_async_copy(hl_hbm.at[pl.ds(0, H)], buf.at[pl.ds(0, H)], sem).wait()
        pltpu.make_async_copy(hl_hbm.at[pl.ds(0, H)], buf.at[pl.ds(H, H)], sem).wait()

    def phase(base):
        pltpu.sync_copy(
            src_hbm.at[c, pl.ds(s * ROWS_PER_TILE + base, PHASE_ROWS)], srcb)
        pltpu.sync_copy(
            dst_hbm.at[pl.ds(s * ROWS_PER_TILE + base, PHASE_ROWS)], dstb)
        issue(0, gbuf0, sem0)
        issue(1, gbuf1, sem1)

        def body(i, _):
            j0 = 2 * i
            drain(gbuf0, sem0)
            pltpu.sync_copy(gbuf0, acc_sh.at[dstb.at[j0]], add=True)

            @pl.when(i < PHASE_ROWS // 2 - 1)
            def _():
                issue(j0 + 2, gbuf0, sem0)

            drain(gbuf1, sem1)
            pltpu.sync_copy(gbuf1, acc_sh.at[dstb.at[j0 + 1]], add=True)

            @pl.when(i < PHASE_ROWS // 2 - 1)
            def _():
                issue(j0 + 3, gbuf1, sem1)

            return _

        lax.fori_loop(0, PHASE_ROWS // 2, body, None)

    phase(0)
    phase(PHASE_ROWS)
    plsc.subcore_barrier()

    @pl.when(c == 0)
    def _():
        pltpu.sync_copy(
            acc_sh.at[pl.ds(s * NODES_PER_TILE, NODES_PER_TILE)],
            out_hbm.at[0, pl.ds(s * NODES_PER_TILE, NODES_PER_TILE)],
        )

    @pl.when(c == 1)
    def _():
        pltpu.sync_copy(
            acc_sh.at[pl.ds(s * NODES_PER_TILE, NODES_PER_TILE)],
            out_hbm.at[1, pl.ds(s * NODES_PER_TILE, NODES_PER_TILE)],
        )


_BLK = 5120
_GRID = NP // _BLK


def _pre_body(x_ref, w_ref, hl_ref, hr_ref):
    y = jnp.dot(x_ref[...], w_ref[...], preferred_element_type=jnp.float32)
    hl_ref[0] = y[:, :HALF]
    hl_ref[1] = y[:, HALF:D_FEAT]
    hr_ref[...] = y[:, D_FEAT:]


_pre_call = pl.pallas_call(
    _pre_body,
    grid=(_GRID,),
    in_specs=[
        pl.BlockSpec((_BLK, D_FEAT), lambda i: (i, 0)),
        pl.BlockSpec((D_FEAT, 2 * D_FEAT), lambda i: (0, 0)),
    ],
    out_specs=[
        pl.BlockSpec((2, _BLK, HALF), lambda i: (0, i, 0)),
        pl.BlockSpec((_BLK, D_FEAT), lambda i: (i, 0)),
    ],
    out_shape=[
        jax.ShapeDtypeStruct((2, NP, HALF), jnp.float32),
        jax.ShapeDtypeStruct((NP, D_FEAT), jnp.float32),
    ],
)


def _mid_body(agg_ref, cnt_ref, hrp_ref, w_ref, hl_ref, hr_ref):
    inv = 1.0 / jnp.clip(cnt_ref[:, 0:1], 1.0, None)
    mean = jnp.concatenate([agg_ref[0], agg_ref[1]], axis=1) * inv
    h = jnp.maximum(mean + hrp_ref[...], 0.0)
    y = jnp.dot(h, w_ref[...], preferred_element_type=jnp.float32)
    hl_ref[0] = y[:, :HALF]
    hl_ref[1] = y[:, HALF:D_FEAT]
    hr_ref[...] = y[:, D_FEAT:]


_mid_call = pl.pallas_call(
    _mid_body,
    grid=(_GRID,),
    in_specs=[
        pl.BlockSpec((2, _BLK, HALF), lambda i: (0, i, 0)),
        pl.BlockSpec((_BLK, HALF), lambda i: (i, 0)),
        pl.BlockSpec((_BLK, D_FEAT), lambda i: (i, 0)),
        pl.BlockSpec((D_FEAT, 2 * D_FEAT), lambda i: (0, 0)),
    ],
    out_specs=[
        pl.BlockSpec((2, _BLK, HALF), lambda i: (0, i, 0)),
        pl.BlockSpec((_BLK, D_FEAT), lambda i: (i, 0)),
    ],
    out_shape=[
        jax.ShapeDtypeStruct((2, NP, HALF), jnp.float32),
        jax.ShapeDtypeStruct((NP, D_FEAT), jnp.float32),
    ],
)


def _post_body(agg_ref, cnt_ref, hrp_ref, b_ref, w1_ref, w2_ref, bias_ref,
               out_ref, psum, pcnt):
    i = pl.program_id(0)

    @pl.when(i == 0)
    def _():
        psum[...] = jnp.zeros_like(psum)
        pcnt[...] = jnp.zeros_like(pcnt)

    inv = 1.0 / jnp.clip(cnt_ref[:, 0:1], 1.0, None)
    mean = jnp.concatenate([agg_ref[0], agg_ref[1]], axis=1) * inv
    h = jnp.maximum(mean + hrp_ref[...], 0.0)
    b = b_ref[0, 0, :]
    gids = lax.broadcasted_iota(jnp.int32, (N_GRAPHS, _BLK), 0)
    onehot = (gids == b[None, :]).astype(jnp.float32)
    psum[...] += jnp.dot(onehot, h, preferred_element_type=jnp.float32)
    pcnt[...] += jnp.sum(onehot, axis=1, keepdims=True)

    @pl.when(i == _GRID - 1)
    def _():
        pooled = psum[...] / jnp.clip(pcnt[...], 1.0, None)
        z = jnp.maximum(
            jnp.dot(pooled, w1_ref[...], preferred_element_type=jnp.float32), 0.0)
        o = jnp.dot(z, w2_ref[...], preferred_element_type=jnp.float32)
        o = o + bias_ref[0, :][None, :]
        m = jnp.max(o, axis=-1, keepdims=True)
        lse = m + jnp.log(jnp.sum(jnp.exp(o - m), axis=-1, keepdims=True))
        out_ref[...] = o - lse


_post_call = pl.pallas_call(
    _post_body,
    grid=(_GRID,),
    in_specs=[
        pl.BlockSpec((2, _BLK, HALF), lambda i: (0, i, 0)),
        pl.BlockSpec((_BLK, HALF), lambda i: (i, 0)),
        pl.BlockSpec((_BLK, D_FEAT), lambda i: (i, 0)),
        pl.BlockSpec((1, 1, _BLK), lambda i: (i, 0, 0)),
        pl.BlockSpec((D_FEAT, D_FEAT), lambda i: (0, 0)),
        pl.BlockSpec((D_FEAT, N_CLASSES), lambda i: (0, 0)),
        pl.BlockSpec((1, N_CLASSES), lambda i: (0, 0)),
    ],
    out_specs=pl.BlockSpec((N_GRAPHS, N_CLASSES), lambda i: (0, 0)),
    out_shape=jax.ShapeDtypeStruct((N_GRAPHS, N_CLASSES), jnp.float32),
    scratch_shapes=[
        pltpu.VMEM((N_GRAPHS, D_FEAT), jnp.float32),
        pltpu.VMEM((N_GRAPHS, 1), jnp.float32),
    ],
)


def kernel(x, edge_index, batch, W1l, W1r, W2l, W2r, W3l, W3r,
           Wlin1, Wlin2, blin2):
    src = edge_index[0].astype(jnp.int32)
    dst = edge_index[1].astype(jnp.int32)
    pad_e = EP - N_EDGES
    src_p = jnp.concatenate([src, jnp.zeros((pad_e,), jnp.int32)])
    dst_p = jnp.concatenate([dst, jnp.full((pad_e,), DUMMY, jnp.int32)])
    src_lo = src_p.reshape(EROWS, CHUNK)
    src2 = jnp.stack([src_lo, src_lo + NP])
    dst2d = dst_p.reshape(EROWS, CHUNK)
    zeros_blk = jnp.zeros((NODES_PER_TILE, HALF), jnp.float32)
    ones_blk = jnp.ones((CHUNK, HALF), jnp.float32)

    x_p = jnp.pad(x, ((0, NP - N_NODES), (0, 0)))
    batch_p = jnp.pad(batch.astype(jnp.int32), (0, NP - N_NODES),
                      constant_values=N_GRAPHS).reshape(_GRID, 1, _BLK)

    w1 = jnp.concatenate([W1l, W1r], axis=1)
    w2 = jnp.concatenate([W2l, W2r], axis=1)
    w3 = jnp.concatenate([W3l, W3r], axis=1)
    bias2 = blin2.reshape(1, N_CLASSES)

    count_kernel, agg_kernel = _sc_kernels()
    cnt = count_kernel(dst2d, ones_blk, zeros_blk)

    hl1, hr1 = _pre_call(x_p, w1)
    agg1 = agg_kernel(hl1.reshape(2 * NP, HALF), src2, dst2d, zeros_blk)
    hl2, hr2 = _mid_call(agg1, cnt, hr1, w2)
    agg2 = agg_kernel(hl2.reshape(2 * NP, HALF), src2, dst2d, zeros_blk)
    hl3, hr3 = _mid_call(agg2, cnt, hr2, w3)
    agg3 = agg_kernel(hl3.reshape(2 * NP, HALF), src2, dst2d, zeros_blk)

    return _post_call(agg3, cnt, hr3, batch_p, Wlin1, Wlin2, bias2)
